# baseline (speedup 1.0000x reference)
import jax
import jax.numpy as jnp
from jax.experimental import pallas as pl

N = 4096
F = 128
H = 2
C = 64
D = 64
HC = H * C
R = 1024


def _feat_body(x_ref, wk_ref, ss4_ref, sn4_ref, sn4t_ref,
               aug0_ref, aug1_ref, afse_ref, afne_ref, selfe_ref):
    xk = jnp.dot(x_ref[...], wk_ref[...], preferred_element_type=jnp.float32)
    af4 = jnp.dot(xk, ss4_ref[...], preferred_element_type=jnp.float32)
    afse_ref[...] = jnp.exp(af4).astype(jnp.bfloat16)
    afn4 = jax.lax.dot_general(sn4t_ref[...], xk, (((1,), (1,)), ((), ())),
                               preferred_element_type=jnp.float32)
    afne_ref[...] = jnp.concatenate(
        [jnp.exp(afn4), jnp.zeros((12, N), jnp.float32)],
        axis=0).astype(jnp.bfloat16)
    ones = jnp.ones((N, 1), jnp.float32)
    aug0_ref[...] = jnp.concatenate([xk[:, :C], ones], axis=1).astype(jnp.bfloat16)
    aug1_ref[...] = jnp.concatenate([xk[:, C:], ones], axis=1).astype(jnp.bfloat16)
    afn_n4 = jnp.dot(xk, sn4_ref[...], preferred_element_type=jnp.float32)
    lg4 = af4 + afn_n4
    selfe_ref[...] = jnp.maximum(jnp.exp(lg4[:, :H]), jnp.exp(lg4[:, H:]))


def _gat_gru_body(a_ref, aug0f_ref, aug1f_ref, aug0b_ref, aug1b_ref,
                  afse_ref, afne_ref, selfe_ref, h_ref,
                  bu_ref, br_ref, bc_ref, gb0_ref, gb1_ref,
                  wu_ref, wr_ref, wc_ref, h1_ref):
    i = pl.program_id(0)
    a_bf = a_ref[...].astype(jnp.bfloat16)
    rr = jax.lax.broadcasted_iota(jnp.int32, (R, R), 0)
    cc = jax.lax.broadcasted_iota(jnp.int32, (R, R), 1)
    a_win = a_ref[:, pl.ds(i * R, R)]
    d = jnp.sum(jnp.where(rr == cc, a_win, 0.0), axis=1, keepdims=True)
    coef = (1.0 - d) * selfe_ref[...]
    convs = []
    for h, augf_ref, augb_ref, gb_ref in (
            (0, aug0f_ref, aug0b_ref, gb0_ref),
            (1, aug1f_ref, aug1b_ref, gb1_ref)):
        p1 = afse_ref[:, h:h + 1]
        p2 = afse_ref[:, 2 + h:3 + h]
        q1 = afne_ref[h:h + 1, :]
        q2 = afne_ref[2 + h:3 + h, :]
        w = a_bf * jnp.maximum(p1 * q1, p2 * q2)
        agg = jnp.dot(w, augf_ref[...], preferred_element_type=jnp.float32)
        agg = agg + coef[:, h:h + 1] * augb_ref[...].astype(jnp.float32)
        convs.append(agg[:, :C] / agg[:, C:C + 1] + gb_ref[...])
    c0, c1 = convs

    h_b = h_ref[...]
    wu = wu_ref[...]
    wr = wr_ref[...]
    wc = wc_ref[...]

    def mm3(w, a0, a1, a2):
        return (jnp.dot(a0, w[:C, :], preferred_element_type=jnp.float32)
                + jnp.dot(a1, w[C:HC, :], preferred_element_type=jnp.float32)
                + jnp.dot(a2, w[HC:, :], preferred_element_type=jnp.float32))

    u = jax.nn.sigmoid(bu_ref[...] + mm3(wu, c0, c1, h_b))
    r = jax.nn.sigmoid(br_ref[...] + mm3(wr, c0, c1, h_b))
    c = jnp.tanh(bc_ref[...] + mm3(wc, c0, c1, r * h_b))
    h1_ref[...] = u * h_b + (1.0 - u) * c


def _decode_body(hb_ref, hf_ref, rp_ref, a_ref):
    hr = jnp.dot(hb_ref[...], rp_ref[...], preferred_element_type=jnp.float32)
    a_ref[...] = jax.lax.dot_general(
        hr, hf_ref[...], (((1,), (1,)), ((), ())),
        preferred_element_type=jnp.float32)


@jax.jit
def kernel(x, a, h_state, kernel, attn_self, attn_neighs, gat_bias,
           b_u, b_r, b_c, W_u, W_r, W_c, R_p):
    x2 = x.reshape(N, F)
    a2 = a.reshape(N, N)
    h2 = h_state.reshape(N, D)
    wk = kernel.reshape(F, HC)
    hsel = (jnp.arange(HC, dtype=jnp.int32) // C)[:, None] \
        == jnp.arange(H, dtype=jnp.int32)[None, :]
    ss = jnp.where(hsel, jnp.tile(attn_self[:, :, 0], (H, 1)), 0.0)
    sn = jnp.where(hsel, jnp.tile(attn_neighs[:, :, 0], (H, 1)), 0.0)
    ss4 = jnp.concatenate([ss, 0.2 * ss], axis=1)
    sn4 = jnp.concatenate([sn, 0.2 * sn], axis=1)
    sn4t = jnp.concatenate([sn.T, 0.2 * sn.T], axis=0)
    gb0 = gat_bias[:C].reshape(1, C)
    gb1 = gat_bias[C:].reshape(1, C)

    aug0, aug1, afse, afne, selfe = pl.pallas_call(
        _feat_body,
        out_shape=(jax.ShapeDtypeStruct((N, C + 1), jnp.bfloat16),
                   jax.ShapeDtypeStruct((N, C + 1), jnp.bfloat16),
                   jax.ShapeDtypeStruct((N, 4), jnp.bfloat16),
                   jax.ShapeDtypeStruct((16, N), jnp.bfloat16),
                   jax.ShapeDtypeStruct((N, H), jnp.float32)),
    )(x2, wk, ss4, sn4, sn4t)

    nblk = N // R
    full = lambda i: (0, 0)
    blk = lambda i: (i, 0)
    h1 = pl.pallas_call(
        _gat_gru_body,
        grid=(nblk,),
        in_specs=[
            pl.BlockSpec((R, N), blk),
            pl.BlockSpec((N, C + 1), full),
            pl.BlockSpec((N, C + 1), full),
            pl.BlockSpec((R, C + 1), blk),
            pl.BlockSpec((R, C + 1), blk),
            pl.BlockSpec((R, 4), blk),
            pl.BlockSpec((16, N), full),
            pl.BlockSpec((R, H), blk),
            pl.BlockSpec((R, D), blk),
            pl.BlockSpec((R, 1), blk),
            pl.BlockSpec((R, 1), blk),
            pl.BlockSpec((R, 1), blk),
            pl.BlockSpec((1, C), full),
            pl.BlockSpec((1, C), full),
            pl.BlockSpec((HC + D, D), full),
            pl.BlockSpec((HC + D, D), full),
            pl.BlockSpec((HC + D, D), full),
        ],
        out_specs=pl.BlockSpec((R, D), blk),
        out_shape=jax.ShapeDtypeStruct((N, D), jnp.float32),
    )(a2, aug0, aug1, aug0, aug1, afse, afne, selfe, h2,
      b_u, b_r, b_c, gb0, gb1, W_u, W_r, W_c)

    A = pl.pallas_call(
        _decode_body,
        grid=(nblk,),
        in_specs=[
            pl.BlockSpec((R, D), blk),
            pl.BlockSpec((N, D), full),
            pl.BlockSpec((D, D), full),
        ],
        out_specs=pl.BlockSpec((R, N), blk),
        out_shape=jax.ShapeDtypeStruct((N, N), jnp.float32),
    )(h1, h1, R_p)

    return (A.reshape(1, N, N), h1.reshape(1, N, D))


# speedup vs baseline: 1.0444x; 1.0444x over previous
import jax
import jax.numpy as jnp
from jax.experimental import pallas as pl
from jax.experimental.pallas import tpu as pltpu

N = 4096
F = 128
H = 2
C = 64
D = 64
HC = H * C
R = 512


def _gat_gru_body(x_ref, wk_ref, ss4_ref, sn4_ref, sn4t_ref, a_ref, h_ref,
                  bu_ref, br_ref, bc_ref, gb0_ref, gb1_ref,
                  wu_ref, wr_ref, wc_ref, h1_ref,
                  aug0_s, aug1_s, afse_s, afne_s, selfe_s):
    i = pl.program_id(0)

    @pl.when(i == 0)
    def _():
        xk = jnp.dot(x_ref[...], wk_ref[...], preferred_element_type=jnp.float32)
        af4 = jnp.dot(xk, ss4_ref[...], preferred_element_type=jnp.float32)
        afse_s[...] = jnp.exp(af4).astype(jnp.bfloat16)
        afn4 = jax.lax.dot_general(sn4t_ref[...], xk, (((1,), (1,)), ((), ())),
                                   preferred_element_type=jnp.float32)
        afne_s[...] = jnp.concatenate(
            [jnp.exp(afn4), jnp.zeros((12, N), jnp.float32)],
            axis=0).astype(jnp.bfloat16)
        ones = jnp.ones((N, 1), jnp.float32)
        aug0_s[...] = jnp.concatenate([xk[:, :C], ones], axis=1).astype(jnp.bfloat16)
        aug1_s[...] = jnp.concatenate([xk[:, C:], ones], axis=1).astype(jnp.bfloat16)
        afn_n4 = jnp.dot(xk, sn4_ref[...], preferred_element_type=jnp.float32)
        lg4 = af4 + afn_n4
        selfe_s[...] = jnp.maximum(jnp.exp(lg4[:, :H]), jnp.exp(lg4[:, H:]))

    a_bf = a_ref[...].astype(jnp.bfloat16)
    rr = jax.lax.broadcasted_iota(jnp.int32, (R, R), 0)
    cc = jax.lax.broadcasted_iota(jnp.int32, (R, R), 1)
    a_win = a_ref[:, pl.ds(i * R, R)]
    d = jnp.sum(jnp.where(rr == cc, a_win, 0.0), axis=1, keepdims=True)
    coef = (1.0 - d) * selfe_s[pl.ds(i * R, R), :]
    convs = []
    for h, aug_s, gb_ref in ((0, aug0_s, gb0_ref), (1, aug1_s, gb1_ref)):
        p1 = afse_s[pl.ds(i * R, R), h:h + 1]
        p2 = afse_s[pl.ds(i * R, R), 2 + h:3 + h]
        q1 = afne_s[h:h + 1, :]
        q2 = afne_s[2 + h:3 + h, :]
        w = a_bf * jnp.maximum(p1 * q1, p2 * q2)
        agg = jnp.dot(w, aug_s[...], preferred_element_type=jnp.float32)
        agg = agg + coef[:, h:h + 1] * aug_s[pl.ds(i * R, R), :].astype(jnp.float32)
        convs.append(agg[:, :C] / agg[:, C:C + 1] + gb_ref[...])
    c0, c1 = convs

    h_b = h_ref[...]
    wu = wu_ref[...]
    wr = wr_ref[...]
    wc = wc_ref[...]

    def mm3(w, a0, a1, a2):
        return (jnp.dot(a0, w[:C, :], preferred_element_type=jnp.float32)
                + jnp.dot(a1, w[C:HC, :], preferred_element_type=jnp.float32)
                + jnp.dot(a2, w[HC:, :], preferred_element_type=jnp.float32))

    u = jax.nn.sigmoid(bu_ref[...] + mm3(wu, c0, c1, h_b))
    r = jax.nn.sigmoid(br_ref[...] + mm3(wr, c0, c1, h_b))
    c = jnp.tanh(bc_ref[...] + mm3(wc, c0, c1, r * h_b))
    h1_ref[...] = u * h_b + (1.0 - u) * c


def _decode_body(hb_ref, hf_ref, rp_ref, a_ref):
    hr = jnp.dot(hb_ref[...], rp_ref[...], preferred_element_type=jnp.float32)
    a_ref[...] = jax.lax.dot_general(
        hr, hf_ref[...], (((1,), (1,)), ((), ())),
        preferred_element_type=jnp.float32)


@jax.jit
def kernel(x, a, h_state, kernel, attn_self, attn_neighs, gat_bias,
           b_u, b_r, b_c, W_u, W_r, W_c, R_p):
    x2 = x.reshape(N, F)
    a2 = a.reshape(N, N)
    h2 = h_state.reshape(N, D)
    wk = kernel.reshape(F, HC)
    hsel = (jnp.arange(HC, dtype=jnp.int32) // C)[:, None] \
        == jnp.arange(H, dtype=jnp.int32)[None, :]
    ss = jnp.where(hsel, jnp.tile(attn_self[:, :, 0], (H, 1)), 0.0)
    sn = jnp.where(hsel, jnp.tile(attn_neighs[:, :, 0], (H, 1)), 0.0)
    ss4 = jnp.concatenate([ss, 0.2 * ss], axis=1)
    sn4 = jnp.concatenate([sn, 0.2 * sn], axis=1)
    sn4t = jnp.concatenate([sn.T, 0.2 * sn.T], axis=0)
    gb0 = gat_bias[:C].reshape(1, C)
    gb1 = gat_bias[C:].reshape(1, C)

    nblk = N // R
    full = lambda i: (0, 0)
    blk = lambda i: (i, 0)
    h1 = pl.pallas_call(
        _gat_gru_body,
        grid=(nblk,),
        in_specs=[
            pl.BlockSpec((N, F), full),
            pl.BlockSpec((F, HC), full),
            pl.BlockSpec((HC, 4), full),
            pl.BlockSpec((HC, 4), full),
            pl.BlockSpec((4, HC), full),
            pl.BlockSpec((R, N), blk),
            pl.BlockSpec((R, D), blk),
            pl.BlockSpec((R, 1), blk),
            pl.BlockSpec((R, 1), blk),
            pl.BlockSpec((R, 1), blk),
            pl.BlockSpec((1, C), full),
            pl.BlockSpec((1, C), full),
            pl.BlockSpec((HC + D, D), full),
            pl.BlockSpec((HC + D, D), full),
            pl.BlockSpec((HC + D, D), full),
        ],
        out_specs=pl.BlockSpec((R, D), blk),
        out_shape=jax.ShapeDtypeStruct((N, D), jnp.float32),
        scratch_shapes=[
            pltpu.VMEM((N, C + 1), jnp.bfloat16),
            pltpu.VMEM((N, C + 1), jnp.bfloat16),
            pltpu.VMEM((N, 4), jnp.bfloat16),
            pltpu.VMEM((16, N), jnp.bfloat16),
            pltpu.VMEM((N, H), jnp.float32),
        ],
    )(x2, wk, ss4, sn4, sn4t, a2, h2, b_u, b_r, b_c, gb0, gb1, W_u, W_r, W_c)

    A = pl.pallas_call(
        _decode_body,
        grid=(nblk,),
        in_specs=[
            pl.BlockSpec((R, D), blk),
            pl.BlockSpec((N, D), full),
            pl.BlockSpec((D, D), full),
        ],
        out_specs=pl.BlockSpec((R, N), blk),
        out_shape=jax.ShapeDtypeStruct((N, N), jnp.float32),
    )(h1, h1, R_p)

    return (A.reshape(1, N, N), h1.reshape(1, N, D))


# speedup vs baseline: 1.0810x; 1.0351x over previous
import jax
import jax.numpy as jnp
from jax.experimental import pallas as pl
from jax.experimental.pallas import tpu as pltpu

N = 4096
F = 128
H = 2
C = 64
D = 64
HC = H * C
R = 512
G = N // R


def _body(x_ref, wk_ref, ss4_ref, sn4_ref, sn4t_ref, a_ref, h_ref,
          bu_ref, br_ref, bc_ref, gb0_ref, gb1_ref,
          wu_ref, wr_ref, wc_ref, rp_ref, a_out_ref, h1_ref,
          aug0_s, aug1_s, afse_s, afne_s, selfe_s, h1_s):
    i = pl.program_id(0)

    @pl.when(i == 0)
    def _():
        xk = jnp.dot(x_ref[...], wk_ref[...], preferred_element_type=jnp.float32)
        af4 = jnp.dot(xk, ss4_ref[...], preferred_element_type=jnp.float32)
        afse_s[...] = jnp.exp(af4).astype(jnp.bfloat16)
        afn4 = jax.lax.dot_general(sn4t_ref[...], xk, (((1,), (1,)), ((), ())),
                                   preferred_element_type=jnp.float32)
        afne_s[...] = jnp.concatenate(
            [jnp.exp(afn4), jnp.zeros((12, N), jnp.float32)],
            axis=0).astype(jnp.bfloat16)
        ones = jnp.ones((N, 1), jnp.float32)
        aug0_s[...] = jnp.concatenate([xk[:, :C], ones], axis=1).astype(jnp.bfloat16)
        aug1_s[...] = jnp.concatenate([xk[:, C:], ones], axis=1).astype(jnp.bfloat16)
        afn_n4 = jnp.dot(xk, sn4_ref[...], preferred_element_type=jnp.float32)
        lg4 = af4 + afn_n4
        selfe_s[...] = jnp.maximum(jnp.exp(lg4[:, :H]), jnp.exp(lg4[:, H:]))

    @pl.when(i < G)
    def _():
        a_bf = a_ref[...].astype(jnp.bfloat16)
        rr = jax.lax.broadcasted_iota(jnp.int32, (R, R), 0)
        cc = jax.lax.broadcasted_iota(jnp.int32, (R, R), 1)
        a_win = a_ref[:, pl.ds(i * R, R)]
        d = jnp.sum(jnp.where(rr == cc, a_win, 0.0), axis=1, keepdims=True)
        coef = (1.0 - d) * selfe_s[pl.ds(i * R, R), :]
        convs = []
        for h, aug_s, gb_ref in ((0, aug0_s, gb0_ref), (1, aug1_s, gb1_ref)):
            p1 = afse_s[pl.ds(i * R, R), h:h + 1]
            p2 = afse_s[pl.ds(i * R, R), 2 + h:3 + h]
            q1 = afne_s[h:h + 1, :]
            q2 = afne_s[2 + h:3 + h, :]
            w = a_bf * jnp.maximum(p1 * q1, p2 * q2)
            agg = jnp.dot(w, aug_s[...], preferred_element_type=jnp.float32)
            agg = agg + coef[:, h:h + 1] * aug_s[pl.ds(i * R, R), :].astype(jnp.float32)
            convs.append(agg[:, :C] / agg[:, C:C + 1] + gb_ref[...])
        c0, c1 = convs

        h_b = h_ref[...]
        wu = wu_ref[...]
        wr = wr_ref[...]
        wc = wc_ref[...]

        def mm3(w, a0, a1, a2):
            return (jnp.dot(a0, w[:C, :], preferred_element_type=jnp.float32)
                    + jnp.dot(a1, w[C:HC, :], preferred_element_type=jnp.float32)
                    + jnp.dot(a2, w[HC:, :], preferred_element_type=jnp.float32))

        u = jax.nn.sigmoid(bu_ref[...] + mm3(wu, c0, c1, h_b))
        r = jax.nn.sigmoid(br_ref[...] + mm3(wr, c0, c1, h_b))
        c = jnp.tanh(bc_ref[...] + mm3(wc, c0, c1, r * h_b))
        h1 = u * h_b + (1.0 - u) * c
        h1_ref[...] = h1
        h1_s[pl.ds(i * R, R), :] = h1

    @pl.when(i >= G)
    def _():
        j = i - G
        hb = h1_s[pl.ds(j * R, R), :]
        hr = jnp.dot(hb, rp_ref[...], preferred_element_type=jnp.float32)
        a_out_ref[...] = jax.lax.dot_general(
            hr, h1_s[...], (((1,), (1,)), ((), ())),
            preferred_element_type=jnp.float32)


@jax.jit
def kernel(x, a, h_state, kernel, attn_self, attn_neighs, gat_bias,
           b_u, b_r, b_c, W_u, W_r, W_c, R_p):
    x2 = x.reshape(N, F)
    a2 = a.reshape(N, N)
    h2 = h_state.reshape(N, D)
    wk = kernel.reshape(F, HC)
    hsel = (jnp.arange(HC, dtype=jnp.int32) // C)[:, None] \
        == jnp.arange(H, dtype=jnp.int32)[None, :]
    ss = jnp.where(hsel, jnp.tile(attn_self[:, :, 0], (H, 1)), 0.0)
    sn = jnp.where(hsel, jnp.tile(attn_neighs[:, :, 0], (H, 1)), 0.0)
    ss4 = jnp.concatenate([ss, 0.2 * ss], axis=1)
    sn4 = jnp.concatenate([sn, 0.2 * sn], axis=1)
    sn4t = jnp.concatenate([sn.T, 0.2 * sn.T], axis=0)
    gb0 = gat_bias[:C].reshape(1, C)
    gb1 = gat_bias[C:].reshape(1, C)

    full = lambda i: (0, 0)
    p1 = lambda i: (jnp.minimum(i, G - 1), 0)
    p2 = lambda i: (jnp.maximum(i - G, 0), 0)
    A, h1 = pl.pallas_call(
        _body,
        grid=(2 * G,),
        in_specs=[
            pl.BlockSpec((N, F), full),
            pl.BlockSpec((F, HC), full),
            pl.BlockSpec((HC, 4), full),
            pl.BlockSpec((HC, 4), full),
            pl.BlockSpec((4, HC), full),
            pl.BlockSpec((R, N), p1),
            pl.BlockSpec((R, D), p1),
            pl.BlockSpec((R, 1), p1),
            pl.BlockSpec((R, 1), p1),
            pl.BlockSpec((R, 1), p1),
            pl.BlockSpec((1, C), full),
            pl.BlockSpec((1, C), full),
            pl.BlockSpec((HC + D, D), full),
            pl.BlockSpec((HC + D, D), full),
            pl.BlockSpec((HC + D, D), full),
            pl.BlockSpec((D, D), full),
        ],
        out_specs=(pl.BlockSpec((R, N), p2),
                   pl.BlockSpec((R, D), p1)),
        out_shape=(jax.ShapeDtypeStruct((N, N), jnp.float32),
                   jax.ShapeDtypeStruct((N, D), jnp.float32)),
        scratch_shapes=[
            pltpu.VMEM((N, C + 1), jnp.bfloat16),
            pltpu.VMEM((N, C + 1), jnp.bfloat16),
            pltpu.VMEM((N, 4), jnp.bfloat16),
            pltpu.VMEM((16, N), jnp.bfloat16),
            pltpu.VMEM((N, H), jnp.float32),
            pltpu.VMEM((N, D), jnp.float32),
        ],
    )(x2, wk, ss4, sn4, sn4t, a2, h2, b_u, b_r, b_c, gb0, gb1,
      W_u, W_r, W_c, R_p)

    return (A.reshape(1, N, N), h1.reshape(1, N, D))
